# baseline (speedup 1.0000x reference)
import functools

import jax
import jax.numpy as jnp
from jax.experimental import pallas as pl
from jax.experimental.pallas import tpu as pltpu

_K = 16
_NEG = float(jnp.finfo(jnp.float32).min)
_BIG = 2**30


def _normalize_body(x_ref, o_ref):
    x = x_ref[...]
    n = jnp.sqrt(jnp.sum(x * x, axis=1, keepdims=True))
    o_ref[...] = x / jnp.maximum(n, 1e-12)


_SORT8 = [(0, 1), (2, 3), (0, 2), (1, 3), (1, 2),
          (4, 5), (6, 7), (4, 6), (5, 7), (5, 6),
          (0, 4), (2, 6), (1, 5), (3, 7),
          (2, 4), (3, 5),
          (1, 2), (3, 4), (5, 6)]


def _topk_body(xr_ref, xall_ref, o_ref, *, n, blk_c, k):
    r = xr_ref.shape[0]
    nc = n // blk_c
    ng = blk_c // 128
    xr = xr_ref[...]

    def dot_block(c):
        xc = xall_ref[pl.ds(c * blk_c, blk_c), :]
        return jax.lax.dot_general(xr, xc, (((1,), (1,)), ((), ())),
                                   preferred_element_type=jnp.float32)

    lane = jax.lax.broadcasted_iota(jnp.int32, (r, 128), 1)
    acc_v = jnp.full((r, 128), _NEG, jnp.float32)
    acc_i = jnp.full((r, 128), _BIG, jnp.int32)
    s_next = dot_block(0)
    for c in range(nc):
        s = s_next
        if c + 1 < nc:
            s_next = dot_block(c + 1)
        v = [s[:, g * 128:(g + 1) * 128] for g in range(ng)]
        ci = [lane + (c * blk_c + g * 128) for g in range(ng)]
        for (a, b) in _SORT8:
            swap = (v[b] > v[a]) | ((v[b] == v[a]) & (ci[b] < ci[a]))
            va = jnp.where(swap, v[b], v[a])
            vb = jnp.where(swap, v[a], v[b])
            ca = jnp.where(swap, ci[b], ci[a])
            cb = jnp.where(swap, ci[a], ci[b])
            v[a], v[b], ci[a], ci[b] = va, vb, ca, cb
        for j in range(k):
            m = jnp.max(v[0], axis=1, keepdims=True)
            idx = jnp.min(jnp.where(v[0] == m, ci[0], _BIG), axis=1, keepdims=True)
            sel = lane == (c * k + j)
            acc_v = jnp.where(sel, m, acc_v)
            acc_i = jnp.where(sel, idx, acc_i)
            is_l = ci[0] == idx
            v = [jnp.where(is_l, v[t + 1], v[t]) for t in range(ng - 1)] + [
                jnp.where(is_l, _NEG, v[ng - 1])]
            ci = [jnp.where(is_l, ci[t + 1], ci[t]) for t in range(ng - 1)] + [
                jnp.where(is_l, _BIG, ci[ng - 1])]
    out_lane = jax.lax.broadcasted_iota(jnp.int32, (r, k), 1)
    out = jnp.zeros((r, k), jnp.int32)
    for j in range(k):
        m = jnp.max(acc_v, axis=1, keepdims=True)
        idx = jnp.min(jnp.where(acc_v == m, acc_i, _BIG), axis=1, keepdims=True)
        out = jnp.where(out_lane == j, idx, out)
        acc_v = jnp.where(acc_i == idx, _NEG, acc_v)
    o_ref[...] = out


def _knn(x, *, blk_r, blk_c, interpret=False):
    n, d = x.shape
    xn = pl.pallas_call(
        _normalize_body,
        grid=(n // blk_r,),
        in_specs=[pl.BlockSpec((blk_r, d), lambda i: (i, 0))],
        out_specs=pl.BlockSpec((blk_r, d), lambda i: (i, 0)),
        out_shape=jax.ShapeDtypeStruct((n, d), jnp.float32),
        interpret=interpret,
    )(x)

    body = functools.partial(_topk_body, n=n, blk_c=blk_c, k=_K)
    nn_idx = pl.pallas_call(
        body,
        grid=(n // blk_r,),
        in_specs=[
            pl.BlockSpec((blk_r, d), lambda i: (i, 0)),
            pl.BlockSpec((n, d), lambda i: (0, 0)),
        ],
        out_specs=pl.BlockSpec((blk_r, _K), lambda i: (i, 0)),
        out_shape=jax.ShapeDtypeStruct((n, _K), jnp.int32),
        interpret=interpret,
    )(xn, xn)
    return nn_idx


def kernel(x):
    n = x.shape[0]
    xvec = x.reshape(n, -1)
    return _knn(xvec, blk_r=256, blk_c=1024)


# speedup vs baseline: 1.2640x; 1.2640x over previous
import functools

import jax
import jax.numpy as jnp
from jax.experimental import pallas as pl
from jax.experimental.pallas import tpu as pltpu

_K = 16
_NEG = float(jnp.finfo(jnp.float32).min)
_BIG = 2**30


def _normalize_body(x_ref, o_ref):
    x = x_ref[...]
    n = jnp.sqrt(jnp.sum(x * x, axis=1, keepdims=True))
    o_ref[...] = x / jnp.maximum(n, 1e-12)


def _topk_body(xr_ref, xall_ref, o_ref, cand_v, cand_i, *, n, blk_c, k):
    r = xr_ref.shape[0]
    nc = n // blk_c
    xr = xr_ref[...]

    def dot_block(c):
        xc = xall_ref[pl.ds(c * blk_c, blk_c), :]
        return jax.lax.dot_general(xr, xc, (((1,), (1,)), ((), ())),
                                   preferred_element_type=jnp.float32)

    s_next = dot_block(0)
    for c in range(nc):
        s = s_next
        if c + 1 < nc:
            s_next = dot_block(c + 1)
        col = jax.lax.broadcasted_iota(jnp.int32, (r, blk_c), 1) + jnp.int32(c * blk_c)
        for j in range(k):
            m = jnp.max(s, axis=1, keepdims=True)
            idx = jnp.min(jnp.where(s == m, col, _BIG), axis=1, keepdims=True)
            cand_v[:, c * k + j] = m[:, 0]
            cand_i[:, c * k + j] = idx[:, 0]
            s = jnp.where(col == idx, _NEG, s)
    v = cand_v[...]
    ci = cand_i[...]
    for j in range(k):
        m = jnp.max(v, axis=1, keepdims=True)
        idx = jnp.min(jnp.where(v == m, ci, _BIG), axis=1, keepdims=True)
        o_ref[:, j] = idx[:, 0]
        v = jnp.where(ci == idx, _NEG, v)


def _knn(x, *, blk_r, blk_c, interpret=False):
    n, d = x.shape
    xn = pl.pallas_call(
        _normalize_body,
        grid=(n // blk_r,),
        in_specs=[pl.BlockSpec((blk_r, d), lambda i: (i, 0))],
        out_specs=pl.BlockSpec((blk_r, d), lambda i: (i, 0)),
        out_shape=jax.ShapeDtypeStruct((n, d), jnp.float32),
        interpret=interpret,
    )(x)

    nc = n // blk_c
    body = functools.partial(_topk_body, n=n, blk_c=blk_c, k=_K)
    nn_idx = pl.pallas_call(
        body,
        grid=(n // blk_r,),
        in_specs=[
            pl.BlockSpec((blk_r, d), lambda i: (i, 0)),
            pl.BlockSpec((n, d), lambda i: (0, 0)),
        ],
        out_specs=pl.BlockSpec((blk_r, _K), lambda i: (i, 0)),
        out_shape=jax.ShapeDtypeStruct((n, _K), jnp.int32),
        scratch_shapes=[
            pltpu.VMEM((blk_r, nc * _K), jnp.float32),
            pltpu.VMEM((blk_r, nc * _K), jnp.int32),
        ],
        interpret=interpret,
    )(xn, xn)
    return nn_idx


def kernel(x):
    n = x.shape[0]
    xvec = x.reshape(n, -1)
    return _knn(xvec, blk_r=512, blk_c=1024)


# speedup vs baseline: 1.3620x; 1.0775x over previous
import functools

import jax
import jax.numpy as jnp
from jax.experimental import pallas as pl
from jax.experimental.pallas import tpu as pltpu

_K = 16
_NEG = float(jnp.finfo(jnp.float32).min)
_BIG = 2**30


def _normalize_body(x_ref, o_ref):
    x = x_ref[...]
    n = jnp.sqrt(jnp.sum(x * x, axis=1, keepdims=True))
    o_ref[...] = x / jnp.maximum(n, 1e-12)


def _topk_body(xr_ref, xall_ref, o_ref, cand_v, cand_i, *, n, blk_c, k):
    r = xr_ref.shape[0]
    nc = n // blk_c
    xr = xr_ref[...]

    def dot_block(c):
        xc = xall_ref[pl.ds(c * blk_c, blk_c), :]
        return jax.lax.dot_general(xr, xc, (((1,), (1,)), ((), ())),
                                   preferred_element_type=jnp.float32)

    s_next = dot_block(0)
    for c in range(nc):
        s = s_next
        if c + 1 < nc:
            s_next = dot_block(c + 1)
        col = jax.lax.broadcasted_iota(jnp.int32, (r, blk_c), 1) + jnp.int32(c * blk_c)
        for j in range(k):
            m = jnp.max(s, axis=1, keepdims=True)
            idx = jnp.min(jnp.where(s == m, col, _BIG), axis=1, keepdims=True)
            cand_v[:, c * k + j] = m[:, 0]
            cand_i[:, c * k + j] = idx[:, 0]
            s = jnp.where(col == idx, _NEG, s)
    v = cand_v[...]
    ci = cand_i[...]
    for j in range(k):
        m = jnp.max(v, axis=1, keepdims=True)
        idx = jnp.min(jnp.where(v == m, ci, _BIG), axis=1, keepdims=True)
        o_ref[:, j] = idx[:, 0]
        v = jnp.where(ci == idx, _NEG, v)


def _knn(x, *, blk_r, blk_c, interpret=False):
    n, d = x.shape
    xn = pl.pallas_call(
        _normalize_body,
        grid=(n // blk_r,),
        in_specs=[pl.BlockSpec((blk_r, d), lambda i: (i, 0))],
        out_specs=pl.BlockSpec((blk_r, d), lambda i: (i, 0)),
        out_shape=jax.ShapeDtypeStruct((n, d), jnp.float32),
        interpret=interpret,
    )(x)

    nc = n // blk_c
    body = functools.partial(_topk_body, n=n, blk_c=blk_c, k=_K)
    nn_idx = pl.pallas_call(
        body,
        grid=(n // blk_r,),
        in_specs=[
            pl.BlockSpec((blk_r, d), lambda i: (i, 0)),
            pl.BlockSpec((n, d), lambda i: (0, 0)),
        ],
        out_specs=pl.BlockSpec((blk_r, _K), lambda i: (i, 0)),
        out_shape=jax.ShapeDtypeStruct((n, _K), jnp.int32),
        scratch_shapes=[
            pltpu.VMEM((blk_r, nc * _K), jnp.float32),
            pltpu.VMEM((blk_r, nc * _K), jnp.int32),
        ],
        interpret=interpret,
    )(xn, xn)
    return nn_idx


def kernel(x):
    n = x.shape[0]
    xvec = x.reshape(n, -1)
    return _knn(xvec, blk_r=1024, blk_c=1024)


# speedup vs baseline: 1.4011x; 1.0287x over previous
import functools

import jax
import jax.numpy as jnp
from jax.experimental import pallas as pl
from jax.experimental.pallas import tpu as pltpu

_K = 16
_NEG = float(jnp.finfo(jnp.float32).min)
_BIG = 2**30


def _normalize_body(x_ref, o_ref):
    x = x_ref[...]
    n = jnp.sqrt(jnp.sum(x * x, axis=1, keepdims=True))
    o_ref[...] = x / jnp.maximum(n, 1e-12)


def _topk_body(xr_ref, xall_ref, o_ref, cand_v, cand_i, *, n, blk_c, k):
    r = xr_ref.shape[0]
    nc = n // blk_c
    xr = xr_ref[...]

    def dot_block(c):
        xc = xall_ref[pl.ds(c * blk_c, blk_c), :]
        return jax.lax.dot_general(xr, xc, (((1,), (1,)), ((), ())),
                                   preferred_element_type=jnp.float32)

    s_next = dot_block(0)
    for c in range(nc):
        s = s_next
        if c + 1 < nc:
            s_next = dot_block(c + 1)
        col = jax.lax.broadcasted_iota(jnp.int32, (r, blk_c), 1) + jnp.int32(c * blk_c)
        for j in range(k):
            m = jnp.max(s, axis=1, keepdims=True)
            idx = jnp.min(jnp.where(s == m, col, _BIG), axis=1, keepdims=True)
            cand_v[:, c * k + j] = m[:, 0]
            cand_i[:, c * k + j] = idx[:, 0]
            s = jnp.where(col == idx, _NEG, s)
    v = cand_v[...]
    ci = cand_i[...]
    for j in range(k):
        m = jnp.max(v, axis=1, keepdims=True)
        idx = jnp.min(jnp.where(v == m, ci, _BIG), axis=1, keepdims=True)
        o_ref[:, j] = idx[:, 0]
        v = jnp.where(ci == idx, _NEG, v)


def _knn(x, *, blk_r, blk_c, interpret=False):
    n, d = x.shape
    xn = pl.pallas_call(
        _normalize_body,
        grid=(n // blk_r,),
        in_specs=[pl.BlockSpec((blk_r, d), lambda i: (i, 0))],
        out_specs=pl.BlockSpec((blk_r, d), lambda i: (i, 0)),
        out_shape=jax.ShapeDtypeStruct((n, d), jnp.float32),
        interpret=interpret,
    )(x)

    nc = n // blk_c
    body = functools.partial(_topk_body, n=n, blk_c=blk_c, k=_K)
    nn_idx = pl.pallas_call(
        body,
        grid=(n // blk_r,),
        in_specs=[
            pl.BlockSpec((blk_r, d), lambda i: (i, 0)),
            pl.BlockSpec((n, d), lambda i: (0, 0)),
        ],
        out_specs=pl.BlockSpec((blk_r, _K), lambda i: (i, 0)),
        out_shape=jax.ShapeDtypeStruct((n, _K), jnp.int32),
        scratch_shapes=[
            pltpu.VMEM((blk_r, nc * _K), jnp.float32),
            pltpu.VMEM((blk_r, nc * _K), jnp.int32),
        ],
        interpret=interpret,
    )(xn, xn)
    return nn_idx


def kernel(x):
    n = x.shape[0]
    xvec = x.reshape(n, -1)
    return _knn(xvec, blk_r=2048, blk_c=1024)
